# Initial kernel scaffold; baseline (speedup 1.0000x reference)
import jax
import jax.numpy as jnp
from jax.experimental import pallas as pl


def kernel(x, masks, conv_w, gamma, beta):
    raise NotImplementedError("write your pallas kernel here")


# speedup vs baseline: 1.2067x; 1.2067x over previous
import numpy as np
import jax
import jax.numpy as jnp
from jax import lax
from jax.experimental import pallas as pl
from jax.experimental.pallas import tpu as pltpu

_EPS = 1e-5
_NCLS = 19.0
_CHUNK = 64


def _resize_wmat(in_size: int, out_size: int) -> np.ndarray:
    scale = out_size / in_size
    inv_scale = 1.0 / scale
    kernel_scale = max(inv_scale, 1.0)
    sample_f = (np.arange(out_size, dtype=np.float64) + 0.5) * inv_scale - 0.5
    x = np.abs(sample_f[None, :] - np.arange(in_size, dtype=np.float64)[:, None]) / kernel_scale
    w = np.maximum(0.0, 1.0 - x)
    tot = w.sum(axis=0, keepdims=True)
    w = np.where(np.abs(tot) > 1000 * np.finfo(np.float32).eps,
                 w / np.where(tot != 0.0, tot, 1.0), 0.0)
    w = np.where(((sample_f >= -0.5) & (sample_f <= in_size - 0.5))[None, :], w, 0.0)
    return w.astype(np.float32)


def _san_body(mask_ref, x_ref, wt_ref, w_ref, g_ref, b_ref, conv_ref, out_ref):
    f32 = jnp.float32
    hi = lax.Precision.HIGHEST

    mask = mask_ref[0]
    t = jnp.dot(wt_ref[...], mask, preferred_element_type=f32, precision=hi)
    m2 = jnp.dot(t, w_ref[...], preferred_element_type=f32, precision=hi)

    nchunk = x_ref.shape[1] // _CHUNK
    ssum = jnp.zeros((56, 56), f32)
    smax = jnp.full((56, 56), -jnp.inf, f32)
    for i in range(nchunk):
        xi = x_ref[0, i * _CHUNK:(i + 1) * _CHUNK]
        mid = xi * m2[None]
        ssum = ssum + jnp.sum(mid, axis=0)
        smax = jnp.maximum(smax, jnp.max(mid, axis=0))
    avg = ssum * (1.0 / x_ref.shape[1])

    acc = jnp.zeros((56, 56), f32)
    for c, img in enumerate((avg, smax, m2)):
        row = jnp.concatenate(
            [jnp.zeros((56, 3), f32), img, jnp.zeros((56, 5), f32)], axis=1)
        pad = jnp.concatenate(
            [jnp.zeros((3, 64), f32), row, jnp.zeros((5, 64), f32)], axis=0)
        for dh in range(7):
            for dw in range(7):
                wgt = conv_ref[0, c, dh, dw]
                acc = acc + wgt * lax.slice(pad, (dh, dw), (dh + 56, dw + 56))
    atten = 1.0 / (1.0 + jnp.exp(-acc))
    matten = m2 * atten

    for i in range(nchunk):
        sl = slice(i * _CHUNK, (i + 1) * _CHUNK)
        xi = x_ref[0, sl]
        o = xi * matten[None]
        s1 = jnp.sum(o, axis=(1, 2), keepdims=True)
        s2 = jnp.sum(o * o, axis=(1, 2), keepdims=True)
        mean = s1 * (1.0 / (56.0 * 56.0))
        var = s2 * (1.0 / (56.0 * 56.0)) - mean * mean
        inv = lax.rsqrt(var + _EPS)
        g = g_ref[sl, :][:, :, None]
        b = b_ref[sl, :][:, :, None]
        scale = inv * g
        bias = b - mean * scale
        out_ref[0, sl] = jnp.maximum(o * scale + bias, 0.0)


def kernel(x, masks, conv_w, gamma, beta):
    n, C, H, W = x.shape
    wmat = _resize_wmat(masks.shape[-1], H)
    wt = jnp.asarray(wmat.T)
    w = jnp.asarray(wmat)
    g19 = (gamma.astype(jnp.float32) * _NCLS).reshape(C, 1)
    b19 = (beta.astype(jnp.float32) * _NCLS).reshape(C, 1)

    return pl.pallas_call(
        _san_body,
        grid=(n,),
        in_specs=[
            pl.BlockSpec((1, masks.shape[1], masks.shape[2]), lambda i: (i, 0, 0)),
            pl.BlockSpec((1, C, H, W), lambda i: (i, 0, 0, 0)),
            pl.BlockSpec((H, masks.shape[1]), lambda i: (0, 0)),
            pl.BlockSpec((masks.shape[1], H), lambda i: (0, 0)),
            pl.BlockSpec((C, 1), lambda i: (0, 0)),
            pl.BlockSpec((C, 1), lambda i: (0, 0)),
            pl.BlockSpec(memory_space=pltpu.SMEM),
        ],
        out_specs=pl.BlockSpec((1, C, H, W), lambda i: (i, 0, 0, 0)),
        out_shape=jax.ShapeDtypeStruct(x.shape, x.dtype),
        compiler_params=pltpu.CompilerParams(
            dimension_semantics=("arbitrary",),
        ),
    )(masks, x, wt, w, g19, b19, conv_w)


# speedup vs baseline: 2.0240x; 1.6773x over previous
import numpy as np
import jax
import jax.numpy as jnp
from jax import lax
from jax.experimental import pallas as pl
from jax.experimental.pallas import tpu as pltpu

_EPS = 1e-5
_NCLS = 19.0
_CHUNK = 64


def _resize_wmat(in_size: int, out_size: int) -> np.ndarray:
    scale = out_size / in_size
    inv_scale = 1.0 / scale
    kernel_scale = max(inv_scale, 1.0)
    sample_f = (np.arange(out_size, dtype=np.float64) + 0.5) * inv_scale - 0.5
    x = np.abs(sample_f[None, :] - np.arange(in_size, dtype=np.float64)[:, None]) / kernel_scale
    w = np.maximum(0.0, 1.0 - x)
    tot = w.sum(axis=0, keepdims=True)
    w = np.where(np.abs(tot) > 1000 * np.finfo(np.float32).eps,
                 w / np.where(tot != 0.0, tot, 1.0), 0.0)
    w = np.where(((sample_f >= -0.5) & (sample_f <= in_size - 0.5))[None, :], w, 0.0)
    return w.astype(np.float32)


def _flat_to_2d(v, H):
    return jnp.concatenate([v[:, h * H:(h + 1) * H] for h in range(H)], axis=0)


def _2d_to_flat(a, H):
    return jnp.concatenate([a[h:h + 1, :] for h in range(H)], axis=1)


def _san_body(mask_ref, x_ref, wt_ref, w_ref, g_ref, b_ref, conv_ref, out_ref):
    f32 = jnp.float32
    hi = lax.Precision.HIGHEST
    C = x_ref.shape[1]
    HW = x_ref.shape[2]
    H = 56

    mask = mask_ref[0]
    t = jnp.dot(wt_ref[...], mask, preferred_element_type=f32, precision=hi)
    m2 = jnp.dot(t, w_ref[...], preferred_element_type=f32, precision=hi)
    mflat = _2d_to_flat(m2, H)

    xall = x_ref[0]
    ones = jnp.full((1, C), 1.0 / C, f32)
    xmean = jnp.dot(ones, xall, preferred_element_type=f32, precision=hi)
    xmax = jnp.max(xall, axis=0, keepdims=True)
    avg2 = _flat_to_2d(xmean, H) * m2
    mx2 = _flat_to_2d(xmax, H) * m2

    acc = jnp.zeros((H, H), f32)
    for c, img in enumerate((avg2, mx2, m2)):
        row = jnp.concatenate(
            [jnp.zeros((H, 3), f32), img, jnp.zeros((H, 5), f32)], axis=1)
        pad = jnp.concatenate(
            [jnp.zeros((3, H + 8), f32), row, jnp.zeros((5, H + 8), f32)], axis=0)
        for dh in range(7):
            for dw in range(7):
                wgt = conv_ref[0, c, dh, dw]
                acc = acc + wgt * lax.slice(pad, (dh, dw), (dh + H, dw + H))
    atten = 1.0 / (1.0 + jnp.exp(-acc))
    matten = _2d_to_flat(m2 * atten, H)

    for i in range(C // _CHUNK):
        sl = slice(i * _CHUNK, (i + 1) * _CHUNK)
        o = x_ref[0, sl] * matten
        s1 = jnp.sum(o, axis=1, keepdims=True)
        s2 = jnp.sum(o * o, axis=1, keepdims=True)
        mean = s1 * (1.0 / HW)
        var = s2 * (1.0 / HW) - mean * mean
        scale = lax.rsqrt(var + _EPS) * g_ref[sl, :]
        bias = b_ref[sl, :] - mean * scale
        out_ref[0, sl] = jnp.maximum(o * scale + bias, 0.0)


def kernel(x, masks, conv_w, gamma, beta):
    n, C, H, W = x.shape
    HW = H * W
    wmat = _resize_wmat(masks.shape[-1], H)
    wt = jnp.asarray(wmat.T)
    wm = jnp.asarray(wmat)
    g19 = (gamma.astype(jnp.float32) * _NCLS).reshape(C, 1)
    b19 = (beta.astype(jnp.float32) * _NCLS).reshape(C, 1)
    xf = x.reshape(n, C, HW)

    out = pl.pallas_call(
        _san_body,
        grid=(n,),
        in_specs=[
            pl.BlockSpec((1, masks.shape[1], masks.shape[2]), lambda i: (i, 0, 0)),
            pl.BlockSpec((1, C, HW), lambda i: (i, 0, 0)),
            pl.BlockSpec((H, masks.shape[1]), lambda i: (0, 0)),
            pl.BlockSpec((masks.shape[1], H), lambda i: (0, 0)),
            pl.BlockSpec((C, 1), lambda i: (0, 0)),
            pl.BlockSpec((C, 1), lambda i: (0, 0)),
            pl.BlockSpec(memory_space=pltpu.SMEM),
        ],
        out_specs=pl.BlockSpec((1, C, HW), lambda i: (i, 0, 0)),
        out_shape=jax.ShapeDtypeStruct((n, C, HW), x.dtype),
        compiler_params=pltpu.CompilerParams(
            dimension_semantics=("arbitrary",),
        ),
    )(masks, xf, wt, wm, g19, b19, conv_w)
    return out.reshape(n, C, H, W)


# speedup vs baseline: 2.0248x; 1.0004x over previous
import numpy as np
import jax
import jax.numpy as jnp
from jax import lax
from jax.experimental import pallas as pl
from jax.experimental.pallas import tpu as pltpu

_EPS = 1e-5
_NCLS = 19.0
_CHUNK = 64


def _resize_wmat(in_size: int, out_size: int) -> np.ndarray:
    scale = out_size / in_size
    inv_scale = 1.0 / scale
    kernel_scale = max(inv_scale, 1.0)
    sample_f = (np.arange(out_size, dtype=np.float64) + 0.5) * inv_scale - 0.5
    x = np.abs(sample_f[None, :] - np.arange(in_size, dtype=np.float64)[:, None]) / kernel_scale
    w = np.maximum(0.0, 1.0 - x)
    tot = w.sum(axis=0, keepdims=True)
    w = np.where(np.abs(tot) > 1000 * np.finfo(np.float32).eps,
                 w / np.where(tot != 0.0, tot, 1.0), 0.0)
    w = np.where(((sample_f >= -0.5) & (sample_f <= in_size - 0.5))[None, :], w, 0.0)
    return w.astype(np.float32)


def _flat_to_2d(v, H):
    return jnp.concatenate([v[:, h * H:(h + 1) * H] for h in range(H)], axis=0)


def _2d_to_flat(a, H):
    return jnp.concatenate([a[h:h + 1, :] for h in range(H)], axis=1)


def _san_body(mask_ref, x_ref, wt_ref, w_ref, g_ref, b_ref, conv_ref, out_ref):
    f32 = jnp.float32
    hi = lax.Precision.HIGHEST
    C = x_ref.shape[1]
    HW = x_ref.shape[2]
    H = 56

    mask = mask_ref[0]
    t = jnp.dot(wt_ref[...], mask, preferred_element_type=f32, precision=hi)
    m2 = jnp.dot(t, w_ref[...], preferred_element_type=f32, precision=hi)
    mflat = _2d_to_flat(m2, H)

    xall = x_ref[0]
    ones = jnp.full((1, C), 1.0 / C, f32)
    xmean = jnp.dot(ones, xall, preferred_element_type=f32, precision=hi)
    xmax = jnp.max(xall, axis=0, keepdims=True)
    avg2 = _flat_to_2d(xmean, H) * m2
    mx2 = _flat_to_2d(xmax, H) * m2

    acc = jnp.zeros((H, H), f32)
    for c, img in enumerate((avg2, mx2, m2)):
        row = jnp.concatenate(
            [jnp.zeros((H, 3), f32), img, jnp.zeros((H, 5), f32)], axis=1)
        pad = jnp.concatenate(
            [jnp.zeros((3, H + 8), f32), row, jnp.zeros((5, H + 8), f32)], axis=0)
        for dh in range(7):
            for dw in range(7):
                wgt = conv_ref[0, c, dh, dw]
                acc = acc + wgt * lax.slice(pad, (dh, dw), (dh + H, dw + H))
    atten = 1.0 / (1.0 + jnp.exp(-acc))
    matten = _2d_to_flat(m2 * atten, H)

    for i in range(C // _CHUNK):
        sl = slice(i * _CHUNK, (i + 1) * _CHUNK)
        o = x_ref[0, sl] * matten
        s1 = jnp.sum(o, axis=1, keepdims=True)
        s2 = jnp.sum(o * o, axis=1, keepdims=True)
        mean = s1 * (1.0 / HW)
        var = s2 * (1.0 / HW) - mean * mean
        scale = lax.rsqrt(var + _EPS) * g_ref[sl, :]
        bias = b_ref[sl, :] - mean * scale
        out_ref[0, sl] = jnp.maximum(o * scale + bias, 0.0)


def kernel(x, masks, conv_w, gamma, beta):
    n, C, H, W = x.shape
    HW = H * W
    wmat = _resize_wmat(masks.shape[-1], H)
    wt = jnp.asarray(wmat.T)
    wm = jnp.asarray(wmat)
    g19 = (gamma.astype(jnp.float32) * _NCLS).reshape(C, 1)
    b19 = (beta.astype(jnp.float32) * _NCLS).reshape(C, 1)
    xf = x.reshape(n, C, HW)

    out = pl.pallas_call(
        _san_body,
        grid=(n,),
        in_specs=[
            pl.BlockSpec((1, masks.shape[1], masks.shape[2]), lambda i: (i, 0, 0)),
            pl.BlockSpec((1, C, HW), lambda i: (i, 0, 0)),
            pl.BlockSpec((H, masks.shape[1]), lambda i: (0, 0)),
            pl.BlockSpec((masks.shape[1], H), lambda i: (0, 0)),
            pl.BlockSpec((C, 1), lambda i: (0, 0)),
            pl.BlockSpec((C, 1), lambda i: (0, 0)),
            pl.BlockSpec(memory_space=pltpu.SMEM),
        ],
        out_specs=pl.BlockSpec((1, C, HW), lambda i: (i, 0, 0)),
        out_shape=jax.ShapeDtypeStruct((n, C, HW), x.dtype),
        compiler_params=pltpu.CompilerParams(
            dimension_semantics=("parallel",),
        ),
    )(masks, xf, wt, wm, g19, b19, conv_w)
    return out.reshape(n, C, H, W)
